# baseline (speedup 1.0000x reference)
import functools

import jax
import jax.numpy as jnp
from jax import lax
from jax.experimental import pallas as pl
from jax.experimental.pallas import tpu as pltpu
from jax.experimental.pallas import tpu_sc as plsc

NC = 2
NS = 16
NW = NC * NS
LANES = 16
C = 256


def _make_sc_kernel(B, L, V, D):
    N = B * L
    per_w = N // NW
    assert N % (NW * C) == 0 and per_w % C == 0
    assert L % 2 == 0 and D == 64
    n_chunks = per_w // C
    CP = C // 2
    HL = L // 2
    mesh = plsc.VectorSubcoreMesh(core_axis_name="c", subcore_axis_name="s")

    @functools.partial(
        pl.kernel,
        out_type=jax.ShapeDtypeStruct((N // 2, 2 * D), jnp.float32),
        mesh=mesh,
        scratch_types=[
            pltpu.VMEM((C,), jnp.int32),
            pltpu.VMEM((C, 2 * D), jnp.float32),
            pltpu.VMEM((CP, 2 * D), jnp.float32),
            pltpu.VMEM((HL, 2 * D), jnp.float32),
            pltpu.SemaphoreType.DMA,
        ],
    )
    def k(x_hbm, tok_hbm, pos_hbm, out_hbm, idx_v, tok_v, out_v, pos_v, sem):
        cid = lax.axis_index("c")
        sid = lax.axis_index("s")
        wid = sid * NC + cid
        pltpu.sync_copy(pos_hbm, pos_v)

        def chunk_body(g, carry):
            base = pl.multiple_of(wid * per_w + g * C, C)
            obase = pl.multiple_of(base // 2, CP)
            phase0 = lax.rem(obase, HL)
            pltpu.sync_copy(x_hbm.at[pl.ds(base, C)], idx_v)
            cps = [
                pltpu.async_copy(
                    tok_hbm.at[idx_v.at[pl.ds(o, 128)]],
                    tok_v.at[pl.ds(o, 128)],
                    sem,
                )
                for o in range(0, C, 128)
            ]
            for cp in cps:
                cp.wait()

            def add_body(m, c2):
                p = lax.rem(phase0 + m, HL)
                for h in range(2):
                    for c in range(D // LANES):
                        i = h * (D // LANES) + c
                        sl_out = pl.ds(i * LANES, LANES)
                        sl_tok = pl.ds(c * LANES, LANES)
                        out_v[m, sl_out] = (
                            tok_v[2 * m + h, sl_tok] + pos_v[p, sl_out]
                        )
                return c2

            lax.fori_loop(0, CP, add_body, 0)
            pltpu.sync_copy(out_v, out_hbm.at[pl.ds(obase, CP)])
            return carry

        lax.fori_loop(0, n_chunks, chunk_body, 0)

    return k


def kernel(x, token_table, pos_table):
    B, L = x.shape
    V, D = token_table.shape
    k = _make_sc_kernel(B, L, V, D)
    x_flat = x.reshape(B * L).astype(jnp.int32)
    table_pad = jnp.pad(token_table, ((0, 0), (0, D)))
    pos_pairs = pos_table.reshape(L // 2, 2 * D)
    out = k(x_flat, table_pad, pos_pairs)
    return out.reshape(B, L, D)


# speedup vs baseline: 1.5581x; 1.5581x over previous
import functools

import jax
import jax.numpy as jnp
from jax import lax
from jax.experimental import pallas as pl
from jax.experimental.pallas import tpu as pltpu
from jax.experimental.pallas import tpu_sc as plsc

NC = 2
NS = 16
NW = NC * NS
LANES = 16


def _make_sc_kernel(B, L, V, D):
    assert B % NW == 0 and D == 64
    per_w = B // NW
    splits = []
    off = 0
    while off < L:
        n = min(L - off, 128)
        splits.append((off, n))
        off += n
    mesh = plsc.VectorSubcoreMesh(core_axis_name="c", subcore_axis_name="s")

    @functools.partial(
        pl.kernel,
        out_type=jax.ShapeDtypeStruct((B, L, D), jnp.float32),
        mesh=mesh,
        scratch_types=[
            pltpu.VMEM((L,), jnp.int32),
            pltpu.VMEM((L, 2 * D), jnp.float32),
            pltpu.VMEM((1, L, D), jnp.float32),
            pltpu.VMEM((L, D), jnp.float32),
            pltpu.SemaphoreType.DMA,
        ],
        compiler_params=pltpu.CompilerParams(use_tc_tiling_on_sc=True),
    )
    def k(x_hbm, tok_hbm, pos_hbm, out_hbm, idx_v, tok_v, out_v, pos_v, sem):
        cid = lax.axis_index("c")
        sid = lax.axis_index("s")
        wid = sid * NC + cid
        pltpu.sync_copy(pos_hbm, pos_v)

        def chunk_body(g, carry):
            row = wid * per_w + g
            base = pl.multiple_of(row * L, L)
            pltpu.sync_copy(x_hbm.at[pl.ds(base, L)], idx_v)
            cps = [
                pltpu.async_copy(
                    tok_hbm.at[idx_v.at[pl.ds(o, n)]],
                    tok_v.at[pl.ds(o, n)],
                    sem,
                )
                for (o, n) in splits
            ]
            for cp in cps:
                cp.wait()

            def add_body(l, c2):
                for c in range(D // LANES):
                    sl = pl.ds(c * LANES, LANES)
                    out_v[0, l, sl] = tok_v[l, sl] + pos_v[l, sl]
                return c2

            lax.fori_loop(0, L, add_body, 0)
            pltpu.sync_copy(out_v, out_hbm.at[pl.ds(row, 1)])
            return carry

        lax.fori_loop(0, per_w, chunk_body, 0)

    return k


def kernel(x, token_table, pos_table):
    B, L = x.shape
    V, D = token_table.shape
    k = _make_sc_kernel(B, L, V, D)
    x_flat = x.reshape(B * L).astype(jnp.int32)
    table_pad = jnp.pad(token_table, ((0, 0), (0, D)))
    return k(x_flat, table_pad, pos_table)
